# baseline (speedup 1.0000x reference)
import functools

import jax
import jax.numpy as jnp
from jax import lax
from jax.experimental import pallas as pl
from jax.experimental.pallas import tpu as pltpu
from jax.experimental.pallas import tpu_sc as plsc

B = 4096
S = 12
D = 1024
K = 3

NC = 2
NS = 16
NW = NC * NS

NOUT = B * K
PER_W = NOUT // NW
CHUNK = 48
NCHUNK = PER_W // CHUNK


def _build_sc_gather():
    mesh = plsc.VectorSubcoreMesh(core_axis_name="c", subcore_axis_name="s")

    @functools.partial(
        pl.kernel,
        mesh=mesh,
        out_type=jax.ShapeDtypeStruct((NOUT, D), jnp.float32),
        scratch_types=[
            pltpu.VMEM((NCHUNK, CHUNK), jnp.int32),
            pltpu.VMEM((CHUNK, D), jnp.float32),
            pltpu.VMEM((CHUNK, D), jnp.float32),
            pltpu.SemaphoreType.DMA,
            pltpu.SemaphoreType.DMA,
        ],
    )
    def sc_gather(table_hbm, idx_hbm, out_hbm, idx_v, rows0, rows1, sem0, sem1):
        wid = lax.axis_index("s") * NC + lax.axis_index("c")
        base = wid * PER_W
        pltpu.sync_copy(idx_hbm.at[wid], idx_v)
        bufs = (rows0, rows1)
        sems = (sem0, sem1)
        copies = [pltpu.async_copy(table_hbm.at[idx_v.at[0]], bufs[0], sems[0])]
        for g in range(NCHUNK):
            copies[g].wait()
            if g + 1 < NCHUNK:
                copies.append(
                    pltpu.async_copy(
                        table_hbm.at[idx_v.at[g + 1]],
                        bufs[(g + 1) % 2],
                        sems[(g + 1) % 2],
                    )
                )
            pltpu.sync_copy(bufs[g % 2], out_hbm.at[pl.ds(base + g * CHUNK, CHUNK)])

    return sc_gather


_SC_GATHER = _build_sc_gather()


def kernel(image_latent):
    rand_values = jax.random.uniform(jax.random.key(1), (B, S), dtype=jnp.float32)
    sel = jnp.argsort(rand_values, axis=-1)[:, :K].astype(jnp.int32)
    flat_idx = (jnp.arange(B, dtype=jnp.int32)[:, None] * S + sel).reshape(
        NW, NCHUNK, CHUNK
    )
    table = image_latent.reshape(B * S, D)
    out = _SC_GATHER(table, flat_idx)
    return out.reshape(B, K, D)
